# baseline (speedup 1.0000x reference)
import functools

import jax
import jax.numpy as jnp
from jax import lax
from jax.experimental import pallas as pl
from jax.experimental.pallas import tpu as pltpu
from jax.experimental.pallas import tpu_sc as plsc

_NW = 32
_G = 128


@functools.lru_cache(maxsize=None)
def _make_gather(n_rows: int, vocab: int, embed: int):
    per_w = n_rows // _NW
    n_g = per_w // _G
    assert per_w % _G == 0 and n_rows % _NW == 0

    mesh = plsc.VectorSubcoreMesh(core_axis_name="c", subcore_axis_name="s")

    @functools.partial(
        pl.kernel,
        out_type=jax.ShapeDtypeStruct((n_rows, embed), jnp.float32),
        mesh=mesh,
        scratch_types=[
            pltpu.VMEM((n_g, _G), jnp.int32),
            pltpu.VMEM((_G, embed), jnp.float32),
            pltpu.VMEM((_G, embed), jnp.float32),
            pltpu.SemaphoreType.DMA,
            pltpu.SemaphoreType.DMA,
        ],
        compiler_params=pltpu.CompilerParams(use_tc_tiling_on_sc=False),
    )
    def grab(table_hbm, idx_hbm, out_hbm, idx_v, rows0, rows1, sem0, sem1):
        wid = lax.axis_index("s") * 2 + lax.axis_index("c")
        base = wid * per_w
        pltpu.sync_copy(idx_hbm.at[wid], idx_v)

        pltpu.async_copy(table_hbm.at[idx_v.at[0]], rows0, sem0)

        def body(p, _):
            j = 2 * p
            pltpu.async_copy(table_hbm.at[idx_v.at[j + 1]], rows1, sem1)
            pltpu.make_async_copy(
                table_hbm.at[idx_v.at[j]], rows0, sem0
            ).wait()
            pltpu.sync_copy(rows0, out_hbm.at[pl.ds(base + j * _G, _G)])

            @pl.when(j + 2 < n_g)
            def _():
                pltpu.async_copy(table_hbm.at[idx_v.at[j + 2]], rows0, sem0)

            pltpu.make_async_copy(
                table_hbm.at[idx_v.at[j + 1]], rows1, sem1
            ).wait()
            pltpu.sync_copy(
                rows1, out_hbm.at[pl.ds(base + (j + 1) * _G, _G)]
            )
            return 0

        lax.fori_loop(0, n_g // 2, body, 0)

    return grab


def kernel(input, voc_emb_weight):
    b, l = input.shape
    vocab, embed = voc_emb_weight.shape
    n = b * l
    idx = input.reshape(_NW, n // _NW // _G, _G).astype(jnp.int32)
    grab = _make_gather(n, vocab, embed)
    out = grab(voc_emb_weight, idx)
    return out.reshape(b, l, embed)


# speedup vs baseline: 1.0197x; 1.0197x over previous
import functools

import jax
import jax.numpy as jnp
from jax import lax
from jax.experimental import pallas as pl
from jax.experimental.pallas import tpu as pltpu
from jax.experimental.pallas import tpu_sc as plsc

_NW = 32
_G = 128
_NB = 8
_NF = 5


@functools.lru_cache(maxsize=None)
def _make_gather(n_rows: int, vocab: int, embed: int):
    per_w = n_rows // _NW
    n_g = per_w // _G
    assert per_w % _G == 0 and n_rows % _NW == 0 and n_g > _NB

    mesh = plsc.VectorSubcoreMesh(core_axis_name="c", subcore_axis_name="s")

    @functools.partial(
        pl.kernel,
        out_type=jax.ShapeDtypeStruct((n_rows, embed), jnp.float32),
        mesh=mesh,
        scratch_types=[
            pltpu.VMEM((n_g, _G), jnp.int32),
            pltpu.VMEM((_NB, _G, embed), jnp.float32),
        ]
        + [pltpu.SemaphoreType.DMA] * (2 * _NB),
        compiler_params=pltpu.CompilerParams(use_tc_tiling_on_sc=False),
    )
    def grab(table_hbm, idx_hbm, out_hbm, idx_v, rows_v, *sems):
        g_sem = sems[:_NB]
        o_sem = sems[_NB:]
        wid = lax.axis_index("s") * 2 + lax.axis_index("c")
        base = wid * per_w
        pltpu.sync_copy(idx_hbm.at[wid], idx_v)

        for s in range(_NF):
            pltpu.async_copy(table_hbm.at[idx_v.at[s]], rows_v.at[s], g_sem[s])

        def body(grp, _):
            for s in range(_NB):
                j = grp * _NB + s
                t = (s + _NF) % _NB

                @pl.when(j + _NF < n_g)
                def _():
                    @pl.when(j + _NF >= _NB)
                    def _():
                        pltpu.make_async_copy(
                            rows_v.at[t],
                            out_hbm.at[pl.ds(base, _G)],
                            o_sem[t],
                        ).wait()

                    pltpu.async_copy(
                        table_hbm.at[idx_v.at[j + _NF]], rows_v.at[t], g_sem[t]
                    )

                pltpu.make_async_copy(
                    table_hbm.at[idx_v.at[j]], rows_v.at[s], g_sem[s]
                ).wait()
                pltpu.async_copy(
                    rows_v.at[s], out_hbm.at[pl.ds(base + j * _G, _G)], o_sem[s]
                )
            return 0

        lax.fori_loop(0, n_g // _NB, body, 0)

        for s in range(_NB):
            pltpu.make_async_copy(
                rows_v.at[s], out_hbm.at[pl.ds(base, _G)], o_sem[s]
            ).wait()

    return grab


def kernel(input, voc_emb_weight):
    b, l = input.shape
    vocab, embed = voc_emb_weight.shape
    n = b * l
    idx = input.reshape(_NW, n // _NW // _G, _G).astype(jnp.int32)
    grab = _make_gather(n, vocab, embed)
    out = grab(voc_emb_weight, idx)
    return out.reshape(b, l, embed)
